# baseline (speedup 1.0000x reference)
import functools

import jax
import jax.numpy as jnp
from jax import lax
from jax.experimental import pallas as pl
from jax.experimental.pallas import tpu as pltpu
from jax.experimental.pallas import tpu_sc as plsc

_BATCH = 1024
_SEQ = 200
_DIM = 128

_info = plsc.get_sparse_core_info()
_NC, _NS, _L = _info.num_cores, _info.num_subcores, _info.num_lanes
_NW = _NC * _NS
_ROWS_PER_W = _BATCH // _NW
_NBUF = 4


def _emb_kernel(inp_hbm, word_hbm, pos_hbm, out_hbm,
                idx_v, pos_v, rows_v, gsem, wsem):
    wid = lax.axis_index("s") * _NC + lax.axis_index("c")
    base = wid * _ROWS_PER_W
    half = _SEQ // 2
    npairs = _ROWS_PER_W // 2

    pltpu.sync_copy(pos_hbm, pos_v)
    pltpu.sync_copy(inp_hbm.at[wid, pl.ds(0, 4)], idx_v.at[0])

    def gather_row(t, q):
        s = lax.rem(t, 2)
        r = 2 * t + q
        cb = lax.rem(lax.div(r, 4), 2)
        r4 = lax.rem(r, 4)
        pltpu.async_copy(word_hbm.at[idx_v.at[cb, r4, 0]],
                         rows_v.at[s, q, pl.ds(0, half)], gsem.at[s])
        pltpu.async_copy(word_hbm.at[idx_v.at[cb, r4, 1]],
                         rows_v.at[s, q, pl.ds(half, half)], gsem.at[s])

    def step(t, _):
        @pl.when(t < npairs)
        def _a():
            s = lax.rem(t, 2)
            cb = lax.rem(lax.div(t, 2), 2)

            @pl.when(jnp.logical_and(lax.rem(t, 2) == 0, t > 0))
            def _stage_idx():
                pltpu.sync_copy(inp_hbm.at[wid, pl.ds(2 * t, 4)],
                                idx_v.at[cb])

            @pl.when(t >= 2)
            def _wait_write():
                pltpu.make_async_copy(rows_v.at[s],
                                      out_hbm.at[pl.ds(base, 2)],
                                      wsem.at[s]).wait()

            gather_row(t, 0)
            gather_row(t, 1)

        @pl.when(t >= 1)
        def _b():
            s = lax.rem(t - 1, 2)
            pltpu.make_async_copy(out_hbm.at[pl.ds(base, 2)], rows_v.at[s],
                                  gsem.at[s]).wait()

            def add_row(rr, _):
                vals = [pos_v[rr, pl.ds(c * _L, _L)]
                        for c in range(_DIM // _L)]
                for q in range(2):
                    for c in range(_DIM // _L):
                        plsc.addupdate(
                            rows_v.at[s, q, rr, pl.ds(c * _L, _L)], vals[c])
                return 0

            lax.fori_loop(0, _SEQ, add_row, 0, unroll=2)
            pltpu.async_copy(rows_v.at[s],
                             out_hbm.at[pl.ds(base + 2 * (t - 1), 2)],
                             wsem.at[s])

        return 0

    lax.fori_loop(0, npairs + 1, step, 0)

    for s in range(2):
        pltpu.make_async_copy(rows_v.at[s], out_hbm.at[pl.ds(base, 2)],
                              wsem.at[s]).wait()


def kernel(inputs, word_table, pos_table):
    inp4 = inputs.reshape(_NW, _ROWS_PER_W, 2, _SEQ // 2)
    mesh = plsc.VectorSubcoreMesh(core_axis_name="c", subcore_axis_name="s")
    run = pl.kernel(
        _emb_kernel,
        mesh=mesh,
        out_type=jax.ShapeDtypeStruct((_BATCH, _SEQ, _DIM), jnp.float32),
        scratch_types=[
            pltpu.VMEM((2, 4, 2, _SEQ // 2), jnp.int32),
            pltpu.VMEM((_SEQ, _DIM), jnp.float32),
            pltpu.VMEM((2, 2, _SEQ, _DIM), jnp.float32),
            pltpu.SemaphoreType.DMA((2,)),
            pltpu.SemaphoreType.DMA((2,)),
        ],
    )
    return run(inp4, word_table, pos_table)
